# baseline (speedup 1.0000x reference)
import functools

import jax
import jax.numpy as jnp
from jax import lax
from jax.experimental import pallas as pl
from jax.experimental.pallas import tpu as pltpu
from jax.experimental.pallas import tpu_sc as plsc

N = 10000
D = 256
BN = 1000

E = 160000
NTILE = 16
EPT = E // NTILE
NSLICE = 640
NLAST = N - 15 * NSLICE
CHUNK = 64
NCHUNK = EPT // CHUNK
TAIL = EPT - NCHUNK * CHUNK


DIN = 2613
NKB = 256
NKS = (DIN + NKB - 1) // NKB


def _stem1_body(xt_ref, w_ref, b_ref, y_ref, st_ref):
    k = pl.program_id(0)
    dn = (((0,), (0,)), ((), ()))

    @pl.when(k == 0)
    def _():
        y_ref[...] = jnp.broadcast_to(b_ref[...], (N, D))

    @pl.when(k < NKS - 1)
    def _():
        y_ref[...] += lax.dot_general(
            xt_ref[...], w_ref[...], dn,
            preferred_element_type=jnp.float32)

    @pl.when(k == NKS - 1)
    def _():
        rows = jax.lax.broadcasted_iota(jnp.int32, (NKB, 1), 0)
        valid = rows < (DIN - (NKS - 1) * NKB)
        xb = jnp.where(valid, xt_ref[...], 0.0)
        wb = jnp.where(valid, w_ref[...], 0.0)
        y_ref[...] += lax.dot_general(
            xb, wb, dn, preferred_element_type=jnp.float32)
        y = y_ref[...]
        st_ref[0:1, :] = jnp.sum(y, axis=0, keepdims=True)
        st_ref[1:2, :] = jnp.sum(y * y, axis=0, keepdims=True)


def _stem_mid_body(y_ref, st_in_ref, g_ref, be_ref, w_ref, b_ref, y2_ref, st_ref):
    i = pl.program_id(0)
    m = st_in_ref[0:1, :] / N
    var = st_in_ref[1:2, :] / N - m * m
    scale = g_ref[...] * lax.rsqrt(var + 1e-5)
    y = y_ref[...]
    h = (y - m) * scale + be_ref[...]
    h = jnp.where(h > 0, h, jnp.exp(h) - 1.0)
    y2 = jnp.dot(h, w_ref[...], preferred_element_type=jnp.float32) + b_ref[...]
    y2_ref[...] = y2

    @pl.when(i == 0)
    def _():
        st_ref[...] = jnp.zeros_like(st_ref)

    st_ref[0:1, :] += jnp.sum(y2, axis=0, keepdims=True)
    st_ref[1:2, :] += jnp.sum(y2 * y2, axis=0, keepdims=True)


def _proj_bn_body(y_ref, st_in_ref, g_ref, be_ref, w_ref, as_ref, ad_ref,
                  hg_ref, ss_ref, sd_ref):
    m = st_in_ref[0:1, :] / N
    var = st_in_ref[1:2, :] / N - m * m
    scale = g_ref[...] * lax.rsqrt(var + 1e-5)
    h = (y_ref[...] - m) * scale + be_ref[...]
    h = jnp.where(h > 0, h, jnp.exp(h) - 1.0)
    hg = jnp.dot(h, w_ref[...], preferred_element_type=jnp.float32)
    hg_ref[0] = hg[:, :128]
    hg_ref[1] = hg[:, 128:]
    ss_ref[...] = jnp.sum(hg * as_ref[...], axis=1, keepdims=True)
    sd_ref[...] = jnp.sum(hg * ad_ref[...], axis=1, keepdims=True)


def _proj_bias_body(s0_ref, s1_ref, bg_ref, w_ref, as_ref, ad_ref,
                    hg_ref, ss_ref, sd_ref):
    h = jnp.concatenate([s0_ref[...], s1_ref[...]], axis=1) + bg_ref[...]
    h = jnp.where(h > 0, h, jnp.exp(h) - 1.0)
    hg = jnp.dot(h, w_ref[...], preferred_element_type=jnp.float32)
    hg_ref[0] = hg[:, :128]
    hg_ref[1] = hg[:, 128:]
    ss_ref[...] = jnp.sum(hg * as_ref[...], axis=1, keepdims=True)
    sd_ref[...] = jnp.sum(hg * ad_ref[...], axis=1, keepdims=True)


def _final_body(s0_ref, s1_ref, bg_ref, o_ref):
    h = jnp.concatenate([s0_ref[...], s1_ref[...]], axis=1) + bg_ref[...]
    o_ref[...] = jnp.where(h > 0, h, jnp.exp(h) - 1.0)


def _row_spec(cols):
    return pl.BlockSpec((BN, cols), lambda i: (i, 0))


def _full_spec(shape):
    return pl.BlockSpec(shape, lambda i: tuple(0 for _ in shape))


def _stem1(x, W1, b1):
    xt = x.T
    return pl.pallas_call(
        _stem1_body,
        grid=(NKS,),
        in_specs=[
            pl.BlockSpec((NKB, N), lambda k: (k, 0)),
            pl.BlockSpec((NKB, D), lambda k: (k, 0)),
            _full_spec((1, D)),
        ],
        out_specs=[_full_spec((N, D)), _full_spec((2, D))],
        out_shape=[
            jax.ShapeDtypeStruct((N, D), jnp.float32),
            jax.ShapeDtypeStruct((2, D), jnp.float32),
        ],
        compiler_params=pltpu.CompilerParams(
            vmem_limit_bytes=100 * 1024 * 1024),
    )(xt, W1, b1.reshape(1, D))


def _stem_mid(y, st, g, be, W, b):
    return pl.pallas_call(
        _stem_mid_body,
        grid=(N // BN,),
        in_specs=[
            _row_spec(D),
            _full_spec((2, D)),
            _full_spec((1, D)),
            _full_spec((1, D)),
            _full_spec((D, D)),
            _full_spec((1, D)),
        ],
        out_specs=[_row_spec(D), _full_spec((2, D))],
        out_shape=[
            jax.ShapeDtypeStruct((N, D), jnp.float32),
            jax.ShapeDtypeStruct((2, D), jnp.float32),
        ],
    )(y, st, g.reshape(1, D), be.reshape(1, D), W, b.reshape(1, D))


def _proj_bn(y, st, g, be, W, a_s, a_d):
    return pl.pallas_call(
        _proj_bn_body,
        grid=(N // BN,),
        in_specs=[
            _row_spec(D),
            _full_spec((2, D)),
            _full_spec((1, D)),
            _full_spec((1, D)),
            _full_spec((D, D)),
            _full_spec((1, D)),
            _full_spec((1, D)),
        ],
        out_specs=[pl.BlockSpec((2, BN, 128), lambda i: (0, i, 0)),
                   _row_spec(1), _row_spec(1)],
        out_shape=[
            jax.ShapeDtypeStruct((2, N, 128), jnp.float32),
            jax.ShapeDtypeStruct((N, 1), jnp.float32),
            jax.ShapeDtypeStruct((N, 1), jnp.float32),
        ],
    )(y, st, g.reshape(1, D), be.reshape(1, D), W, a_s.reshape(1, D),
      a_d.reshape(1, D))


def _proj_bias(s0, s1, bg, W, a_s, a_d):
    return pl.pallas_call(
        _proj_bias_body,
        grid=(N // BN,),
        in_specs=[
            _row_spec(128),
            _row_spec(128),
            _full_spec((1, D)),
            _full_spec((D, D)),
            _full_spec((1, D)),
            _full_spec((1, D)),
        ],
        out_specs=[pl.BlockSpec((2, BN, 128), lambda i: (0, i, 0)),
                   _row_spec(1), _row_spec(1)],
        out_shape=[
            jax.ShapeDtypeStruct((2, N, 128), jnp.float32),
            jax.ShapeDtypeStruct((N, 1), jnp.float32),
            jax.ShapeDtypeStruct((N, 1), jnp.float32),
        ],
    )(s0, s1, bg.reshape(1, D), W, a_s.reshape(1, D), a_d.reshape(1, D))


def _final(s0, s1, bg):
    return pl.pallas_call(
        _final_body,
        grid=(N // BN,),
        in_specs=[_row_spec(128), _row_spec(128), _full_spec((1, D))],
        out_specs=_row_spec(D),
        out_shape=jax.ShapeDtypeStruct((N, D), jnp.float32),
    )(s0, s1, bg.reshape(1, D))


def _edge_a_body(ss_h, sd_h, pk_h, ex_out, den_out,
                 bufA, bufB, pk_v, ex_v, didxA, didx16, zed_v, denom_sh):
    c = lax.axis_index("c")
    s = lax.axis_index("s")
    ebase = pl.multiple_of(s * EPT, 8)
    nbase = pl.multiple_of(s * NSLICE, 8)
    zero16 = jnp.zeros((16,), jnp.float32)

    pltpu.sync_copy(ss_h, bufA)
    pltpu.sync_copy(sd_h, bufB)
    pltpu.sync_copy(pk_h.at[pl.ds(ebase, EPT)], pk_v)

    def zl(i, _):
        zed_v[pl.ds(i * 16, 16)] = zero16
        return 0
    lax.fori_loop(0, NSLICE // 16, zl, 0)

    @pl.when(s < NTILE - 1)
    def _():
        pltpu.sync_copy(zed_v, denom_sh.at[pl.ds(nbase, NSLICE)])

    @pl.when(s == NTILE - 1)
    def _():
        pltpu.sync_copy(zed_v.at[pl.ds(0, NLAST)],
                        denom_sh.at[pl.ds(nbase, NLAST)])

    plsc.subcore_barrier()

    def grp(gidx, didx_ref, slot):
        off = gidx * 16
        pk = pk_v[pl.ds(off, 16)]
        sidx = pk >> 16
        didx = pk & 0xFFFF
        a = (plsc.load_gather(bufA, [sidx])
             + plsc.load_gather(bufB, [didx]))
        a = jnp.where(a >= 0, a, 0.2 * a)
        ex_v[pl.ds(off, 16)] = jnp.exp(a)
        didx_ref[pl.ds(slot * 16, 16)] = didx

    def chunk(k, _):
        def g(j, _):
            grp(k * 8 + j, didxA, j)
            return 0
        lax.fori_loop(0, 8, g, 0)
        pltpu.sync_copy(ex_v.at[pl.ds(pl.multiple_of(k * 128, 8), 128)],
                        denom_sh.at[didxA], add=True)
        return 0
    lax.fori_loop(0, (EPT // 16) // 8, chunk, 0)

    grp((EPT // 16) - 1, didx16, 0)
    pltpu.sync_copy(ex_v.at[pl.ds(EPT - 16, 16)],
                    denom_sh.at[didx16], add=True)

    @pl.when(c == 0)
    def _():
        pltpu.sync_copy(ex_v, ex_out.at[pl.ds(ebase, EPT)])

    plsc.subcore_barrier()

    @pl.when((c == 0) & (s < NTILE - 1))
    def _():
        pltpu.sync_copy(denom_sh.at[pl.ds(nbase, NSLICE)], zed_v)
        pltpu.sync_copy(zed_v, den_out.at[pl.ds(nbase, NSLICE)])

    @pl.when((c == 0) & (s == NTILE - 1))
    def _():
        pltpu.sync_copy(denom_sh.at[pl.ds(nbase, NLAST)],
                        zed_v.at[pl.ds(0, NLAST)])
        pltpu.sync_copy(zed_v.at[pl.ds(0, NLAST)],
                        den_out.at[pl.ds(nbase, NLAST)])


def _edge_b_body(hgr, pk_h, ex_h, den_h, out_h,
                 pk_v, ex_v, sidx3, didx3, rows3, rows16,
                 sidx16, didx16, den_v, gsem, ssem, acc_sh):
    c = lax.axis_index("c")
    s = lax.axis_index("s")
    ebase = pl.multiple_of(s * EPT, 8)
    nbase = pl.multiple_of(s * NSLICE, 8)

    pltpu.sync_copy(pk_h.at[pl.ds(ebase, EPT)], pk_v)
    pltpu.sync_copy(ex_h.at[pl.ds(ebase, EPT)], ex_v)

    zero16 = jnp.zeros((16,), jnp.float32)

    def zr(r, _):
        for j in range(8):
            rows3[0, r, pl.ds(j * 16, 16)] = zero16
        return 0
    lax.fori_loop(0, CHUNK, zr, 0)

    @pl.when(s < NTILE - 1)
    def _():
        for k in range(NSLICE // CHUNK):
            pltpu.sync_copy(rows3.at[0],
                            acc_sh.at[pl.ds(nbase + k * CHUNK, CHUNK)])

    @pl.when(s == NTILE - 1)
    def _():
        for k in range(NLAST // CHUNK):
            pltpu.sync_copy(rows3.at[0],
                            acc_sh.at[pl.ds(nbase + k * CHUNK, CHUNK)])
        pltpu.sync_copy(rows3.at[0, pl.ds(0, NLAST % CHUNK)],
                        acc_sh.at[pl.ds(nbase + (NLAST // CHUNK) * CHUNK,
                                        NLAST % CHUNK)])

    plsc.subcore_barrier()

    cN = jnp.full((16,), c * N, jnp.int32)

    def build(g1, u):
        def bj(j, _):
            off = g1 * CHUNK + j * 16
            pk = pk_v[pl.ds(off, 16)]
            sidx3[u, pl.ds(j * 16, 16)] = (pk >> 16) + cN
            didx3[u, pl.ds(j * 16, 16)] = pk & 0xFFFF
            return 0
        lax.fori_loop(0, CHUNK // 16, bj, 0)

    def weight(g0, u):
        def wr(i, _):
            for w in range(2):
                r = i * 2 + w
                splat = plsc.load_gather(
                    ex_v, [jnp.full((16,), g0 * CHUNK + r, jnp.int32)])
                for j in range(8):
                    rows3[u, r, pl.ds(j * 16, 16)] = (
                        rows3[u, r, pl.ds(j * 16, 16)] * splat)
            return 0
        lax.fori_loop(0, CHUNK // 2, wr, 0)

    def sctr(u):
        return pltpu.async_copy(rows3.at[u], acc_sh.at[didx3.at[u]], ssem,
                                add=True)

    def gthr(u):
        return pltpu.async_copy(hgr.at[sidx3.at[u]], rows3.at[u], gsem)

    def gwait(u):
        pltpu.make_async_copy(hgr.at[sidx3.at[u]], rows3.at[u], gsem).wait()

    def swait(u):
        pltpu.make_async_copy(rows3.at[u], acc_sh.at[didx3.at[u]],
                              ssem).wait()

    build(0, 0)
    gthr(0)

    def triple(t, _):
        for u in range(3):
            g = 3 * t + u
            gwait(u)
            nu = (u + 1) % 3

            @pl.when(g < NCHUNK - 1)
            def _():
                @pl.when(g >= 2)
                def _():
                    swait(nu)
                build(g + 1, nu)
                gthr(nu)
            weight(g, u)
            sctr(u)
        return 0
    lax.fori_loop(0, NCHUNK // 3, triple, 0)
    swait((NCHUNK - 2) % 3)
    swait((NCHUNK - 1) % 3)

    toff = NCHUNK * CHUNK
    pk = pk_v[pl.ds(toff, 16)]
    sidx16[...] = (pk >> 16) + cN
    didx16[...] = pk & 0xFFFF
    pltpu.async_copy(hgr.at[sidx16], rows16, gsem).wait()

    def wrt(r, _):
        splat = plsc.load_gather(ex_v, [jnp.full((16,), toff + r, jnp.int32)])
        for j in range(8):
            rows16[r, pl.ds(j * 16, 16)] = rows16[r, pl.ds(j * 16, 16)] * splat
        return 0
    lax.fori_loop(0, TAIL, wrt, 0)
    pltpu.sync_copy(rows16, acc_sh.at[didx16], add=True)

    plsc.subcore_barrier()

    def flush(nslice):
        pltpu.sync_copy(den_h.at[pl.ds(nbase, nslice)],
                        den_v.at[pl.ds(0, nslice)])
        for k in range(nslice // CHUNK):
            off = nbase + k * CHUNK
            pltpu.sync_copy(acc_sh.at[pl.ds(off, CHUNK)], rows3.at[0])

            def fr(i, _):
                for w in range(2):
                    r = i * 2 + w
                    dn = plsc.load_gather(
                        den_v, [jnp.full((16,), k * CHUNK + r, jnp.int32)])
                    rc = 1.0 / (dn + 1e-16)
                    for j in range(8):
                        rows3[0, r, pl.ds(j * 16, 16)] = (
                            rows3[0, r, pl.ds(j * 16, 16)] * rc)
                return 0
            lax.fori_loop(0, CHUNK // 2, fr, 0)
            pltpu.sync_copy(rows3.at[0], out_h.at[c, pl.ds(off, CHUNK)])

    @pl.when(s < NTILE - 1)
    def _():
        flush(NSLICE)

    @pl.when(s == NTILE - 1)
    def _():
        flush(NLAST - NLAST % CHUNK)
        off = nbase + (NLAST - NLAST % CHUNK)
        pltpu.sync_copy(den_h.at[pl.ds(off, 16)], den_v.at[pl.ds(0, 16)])
        pltpu.sync_copy(acc_sh.at[pl.ds(off, 16)], rows16)

        def fr16(r, _):
            dn = plsc.load_gather(den_v, [jnp.full((16,), r, jnp.int32)])
            rc = 1.0 / (dn + 1e-16)
            for j in range(8):
                rows16[r, pl.ds(j * 16, 16)] = (
                    rows16[r, pl.ds(j * 16, 16)] * rc)
            return 0
        lax.fori_loop(0, 16, fr16, 0)
        pltpu.sync_copy(rows16, out_h.at[c, pl.ds(off, 16)])


def _edge_a(ss, sd, packed):
    return pl.kernel(
        _edge_a_body,
        out_type=[
            jax.ShapeDtypeStruct((E,), jnp.float32),
            jax.ShapeDtypeStruct((N,), jnp.float32),
        ],
        mesh=plsc.VectorSubcoreMesh(core_axis_name="c", subcore_axis_name="s"),
        compiler_params=pltpu.CompilerParams(needs_layout_passes=False),
        scratch_types=[
            pltpu.VMEM((N,), jnp.float32),
            pltpu.VMEM((N,), jnp.float32),
            pltpu.VMEM((EPT,), jnp.int32),
            pltpu.VMEM((EPT,), jnp.float32),
            pltpu.VMEM((128,), jnp.int32),
            pltpu.VMEM((16,), jnp.int32),
            pltpu.VMEM((NSLICE,), jnp.float32),
            pltpu.VMEM_SHARED((N,), jnp.float32),
        ],
    )(ss, sd, packed)


def _edge_b(hgr, packed, ex, den):
    return pl.kernel(
        _edge_b_body,
        out_type=jax.ShapeDtypeStruct((2, N, 128), jnp.float32),
        mesh=plsc.VectorSubcoreMesh(core_axis_name="c", subcore_axis_name="s"),
        compiler_params=pltpu.CompilerParams(needs_layout_passes=False),
        scratch_types=[
            pltpu.VMEM((EPT,), jnp.int32),
            pltpu.VMEM((EPT,), jnp.float32),
            pltpu.VMEM((3, CHUNK), jnp.int32),
            pltpu.VMEM((3, CHUNK), jnp.int32),
            pltpu.VMEM((3, CHUNK, 128), jnp.float32),
            pltpu.VMEM((16, 128), jnp.float32),
            pltpu.VMEM((16,), jnp.int32),
            pltpu.VMEM((16,), jnp.int32),
            pltpu.VMEM((NSLICE,), jnp.float32),
            pltpu.SemaphoreType.DMA,
            pltpu.SemaphoreType.DMA,
            pltpu.VMEM_SHARED((N, 128), jnp.float32),
        ],
    )(hgr, packed, ex, den)


def _edge_phase(hgsplit, ss, sd, packed):
    ex, den = _edge_a(ss, sd, packed)
    return _edge_b(hgsplit.reshape(2 * N, 128), packed, ex, den)


def kernel(x, edges, W1, b1, g1, be1, W2, b2, g2, be2, W3, b3, g3, be3,
           Wg1, as1, ad1, bg1, Wg2, as2, ad2, bg2):
    packed = (edges[0] << 16) | edges[1]
    y1, st1 = _stem1(x, W1, b1)
    y2, st2 = _stem_mid(y1, st1, g1, be1, W2, b2)
    y3, st3 = _stem_mid(y2, st2, g2, be2, W3, b3)
    hg1, ss1, sd1 = _proj_bn(y3, st3, g3, be3, Wg1, as1, ad1)
    s1 = _edge_phase(hg1, ss1[:, 0], sd1[:, 0], packed)
    hg2, ss2, sd2 = _proj_bias(s1[0], s1[1], bg1, Wg2, as2, ad2)
    s2 = _edge_phase(hg2, ss2[:, 0], sd2[:, 0], packed)
    return _final(s2[0], s2[1], bg2)


# speedup vs baseline: 1.0355x; 1.0355x over previous
import functools

import jax
import jax.numpy as jnp
from jax import lax
from jax.experimental import pallas as pl
from jax.experimental.pallas import tpu as pltpu
from jax.experimental.pallas import tpu_sc as plsc

N = 10000
D = 256
BN = 1000

E = 160000
NTILE = 16
EPT = E // NTILE
NSLICE = 640
NLAST = N - 15 * NSLICE
CHUNK = 64
NCHUNK = EPT // CHUNK
TAIL = EPT - NCHUNK * CHUNK


DIN = 2613
NKB = 256
NKS = (DIN + NKB - 1) // NKB


def _stem1_body(xt_ref, w_ref, b_ref, y_ref, st_ref):
    k = pl.program_id(0)
    dn = (((0,), (0,)), ((), ()))

    @pl.when(k == 0)
    def _():
        y_ref[...] = jnp.broadcast_to(b_ref[...], (N, D))

    @pl.when(k < NKS - 1)
    def _():
        y_ref[...] += lax.dot_general(
            xt_ref[...], w_ref[...], dn,
            preferred_element_type=jnp.float32)

    @pl.when(k == NKS - 1)
    def _():
        rows = jax.lax.broadcasted_iota(jnp.int32, (NKB, 1), 0)
        valid = rows < (DIN - (NKS - 1) * NKB)
        xb = jnp.where(valid, xt_ref[...], 0.0)
        wb = jnp.where(valid, w_ref[...], 0.0)
        y_ref[...] += lax.dot_general(
            xb, wb, dn, preferred_element_type=jnp.float32)
        y = y_ref[...]
        st_ref[0:1, :] = jnp.sum(y, axis=0, keepdims=True)
        st_ref[1:2, :] = jnp.sum(y * y, axis=0, keepdims=True)


def _stem_mid_body(y_ref, st_in_ref, g_ref, be_ref, w_ref, b_ref, y2_ref, st_ref):
    i = pl.program_id(0)
    m = st_in_ref[0:1, :] / N
    var = st_in_ref[1:2, :] / N - m * m
    scale = g_ref[...] * lax.rsqrt(var + 1e-5)
    y = y_ref[...]
    h = (y - m) * scale + be_ref[...]
    h = jnp.where(h > 0, h, jnp.exp(h) - 1.0)
    y2 = jnp.dot(h, w_ref[...], preferred_element_type=jnp.float32) + b_ref[...]
    y2_ref[...] = y2

    @pl.when(i == 0)
    def _():
        st_ref[...] = jnp.zeros_like(st_ref)

    st_ref[0:1, :] += jnp.sum(y2, axis=0, keepdims=True)
    st_ref[1:2, :] += jnp.sum(y2 * y2, axis=0, keepdims=True)


def _proj_bn_body(y_ref, st_in_ref, g_ref, be_ref, w_ref, as_ref, ad_ref,
                  hg_ref, ss_ref, sd_ref):
    m = st_in_ref[0:1, :] / N
    var = st_in_ref[1:2, :] / N - m * m
    scale = g_ref[...] * lax.rsqrt(var + 1e-5)
    h = (y_ref[...] - m) * scale + be_ref[...]
    h = jnp.where(h > 0, h, jnp.exp(h) - 1.0)
    hg = jnp.dot(h, w_ref[...], preferred_element_type=jnp.float32)
    hg_ref[0] = hg[:, :128]
    hg_ref[1] = hg[:, 128:]
    ss_ref[...] = jnp.sum(hg * as_ref[...], axis=1, keepdims=True)
    sd_ref[...] = jnp.sum(hg * ad_ref[...], axis=1, keepdims=True)


def _proj_bias_body(s0_ref, s1_ref, bg_ref, w_ref, as_ref, ad_ref,
                    hg_ref, ss_ref, sd_ref):
    h = jnp.concatenate([s0_ref[...], s1_ref[...]], axis=1) + bg_ref[...]
    h = jnp.where(h > 0, h, jnp.exp(h) - 1.0)
    hg = jnp.dot(h, w_ref[...], preferred_element_type=jnp.float32)
    hg_ref[0] = hg[:, :128]
    hg_ref[1] = hg[:, 128:]
    ss_ref[...] = jnp.sum(hg * as_ref[...], axis=1, keepdims=True)
    sd_ref[...] = jnp.sum(hg * ad_ref[...], axis=1, keepdims=True)


def _final_body(s0_ref, s1_ref, bg_ref, o_ref):
    h = jnp.concatenate([s0_ref[...], s1_ref[...]], axis=1) + bg_ref[...]
    o_ref[...] = jnp.where(h > 0, h, jnp.exp(h) - 1.0)


def _row_spec(cols):
    return pl.BlockSpec((BN, cols), lambda i: (i, 0))


def _full_spec(shape):
    return pl.BlockSpec(shape, lambda i: tuple(0 for _ in shape))


def _stem1(x, W1, b1):
    xt = x.T
    return pl.pallas_call(
        _stem1_body,
        grid=(NKS,),
        in_specs=[
            pl.BlockSpec((NKB, N), lambda k: (k, 0)),
            pl.BlockSpec((NKB, D), lambda k: (k, 0)),
            _full_spec((1, D)),
        ],
        out_specs=[_full_spec((N, D)), _full_spec((2, D))],
        out_shape=[
            jax.ShapeDtypeStruct((N, D), jnp.float32),
            jax.ShapeDtypeStruct((2, D), jnp.float32),
        ],
        compiler_params=pltpu.CompilerParams(
            vmem_limit_bytes=100 * 1024 * 1024),
    )(xt, W1, b1.reshape(1, D))


def _stem_mid(y, st, g, be, W, b):
    return pl.pallas_call(
        _stem_mid_body,
        grid=(N // BN,),
        in_specs=[
            _row_spec(D),
            _full_spec((2, D)),
            _full_spec((1, D)),
            _full_spec((1, D)),
            _full_spec((D, D)),
            _full_spec((1, D)),
        ],
        out_specs=[_row_spec(D), _full_spec((2, D))],
        out_shape=[
            jax.ShapeDtypeStruct((N, D), jnp.float32),
            jax.ShapeDtypeStruct((2, D), jnp.float32),
        ],
    )(y, st, g.reshape(1, D), be.reshape(1, D), W, b.reshape(1, D))


def _proj_bn(y, st, g, be, W, a_s, a_d):
    return pl.pallas_call(
        _proj_bn_body,
        grid=(N // BN,),
        in_specs=[
            _row_spec(D),
            _full_spec((2, D)),
            _full_spec((1, D)),
            _full_spec((1, D)),
            _full_spec((D, D)),
            _full_spec((1, D)),
            _full_spec((1, D)),
        ],
        out_specs=[pl.BlockSpec((2, BN, 128), lambda i: (0, i, 0)),
                   _row_spec(1), _row_spec(1)],
        out_shape=[
            jax.ShapeDtypeStruct((2, N, 128), jnp.float32),
            jax.ShapeDtypeStruct((N, 1), jnp.float32),
            jax.ShapeDtypeStruct((N, 1), jnp.float32),
        ],
    )(y, st, g.reshape(1, D), be.reshape(1, D), W, a_s.reshape(1, D),
      a_d.reshape(1, D))


def _proj_bias(s0, s1, bg, W, a_s, a_d):
    return pl.pallas_call(
        _proj_bias_body,
        grid=(N // BN,),
        in_specs=[
            _row_spec(128),
            _row_spec(128),
            _full_spec((1, D)),
            _full_spec((D, D)),
            _full_spec((1, D)),
            _full_spec((1, D)),
        ],
        out_specs=[pl.BlockSpec((2, BN, 128), lambda i: (0, i, 0)),
                   _row_spec(1), _row_spec(1)],
        out_shape=[
            jax.ShapeDtypeStruct((2, N, 128), jnp.float32),
            jax.ShapeDtypeStruct((N, 1), jnp.float32),
            jax.ShapeDtypeStruct((N, 1), jnp.float32),
        ],
    )(s0, s1, bg.reshape(1, D), W, a_s.reshape(1, D), a_d.reshape(1, D))


def _final(s0, s1, bg):
    return pl.pallas_call(
        _final_body,
        grid=(N // BN,),
        in_specs=[_row_spec(128), _row_spec(128), _full_spec((1, D))],
        out_specs=_row_spec(D),
        out_shape=jax.ShapeDtypeStruct((N, D), jnp.float32),
    )(s0, s1, bg.reshape(1, D))


def _edge_a_body(ss_h, sd_h, pk_h, ex_out, den_out,
                 bufA, bufB, pk_v, ex_v, didxA, didx16, zed_v, denom_sh):
    c = lax.axis_index("c")
    s = lax.axis_index("s")
    ebase = pl.multiple_of(s * EPT, 8)
    nbase = pl.multiple_of(s * NSLICE, 8)
    zero16 = jnp.zeros((16,), jnp.float32)

    pltpu.sync_copy(ss_h, bufA)
    pltpu.sync_copy(sd_h, bufB)
    pltpu.sync_copy(pk_h.at[pl.ds(ebase, EPT)], pk_v)

    def zl(i, _):
        zed_v[pl.ds(i * 16, 16)] = zero16
        return 0
    lax.fori_loop(0, NSLICE // 16, zl, 0)

    @pl.when(s < NTILE - 1)
    def _():
        pltpu.sync_copy(zed_v, denom_sh.at[pl.ds(nbase, NSLICE)])

    @pl.when(s == NTILE - 1)
    def _():
        pltpu.sync_copy(zed_v.at[pl.ds(0, NLAST)],
                        denom_sh.at[pl.ds(nbase, NLAST)])

    plsc.subcore_barrier()

    def grp(gidx, didx_ref, slot):
        off = gidx * 16
        pk = pk_v[pl.ds(off, 16)]
        sidx = pk >> 16
        didx = pk & 0xFFFF
        a = (plsc.load_gather(bufA, [sidx])
             + plsc.load_gather(bufB, [didx]))
        a = jnp.where(a >= 0, a, 0.2 * a)
        ex_v[pl.ds(off, 16)] = jnp.exp(a)
        didx_ref[pl.ds(slot * 16, 16)] = didx

    def chunk(k, _):
        def g(j, _):
            grp(k * 8 + j, didxA, j)
            return 0
        lax.fori_loop(0, 8, g, 0)
        pltpu.sync_copy(ex_v.at[pl.ds(pl.multiple_of(k * 128, 8), 128)],
                        denom_sh.at[didxA], add=True)
        return 0
    lax.fori_loop(0, (EPT // 16) // 8, chunk, 0)

    grp((EPT // 16) - 1, didx16, 0)
    pltpu.sync_copy(ex_v.at[pl.ds(EPT - 16, 16)],
                    denom_sh.at[didx16], add=True)

    @pl.when(c == 0)
    def _():
        pltpu.sync_copy(ex_v, ex_out.at[pl.ds(ebase, EPT)])

    plsc.subcore_barrier()

    @pl.when((c == 0) & (s < NTILE - 1))
    def _():
        pltpu.sync_copy(denom_sh.at[pl.ds(nbase, NSLICE)], zed_v)
        pltpu.sync_copy(zed_v, den_out.at[pl.ds(nbase, NSLICE)])

    @pl.when((c == 0) & (s == NTILE - 1))
    def _():
        pltpu.sync_copy(denom_sh.at[pl.ds(nbase, NLAST)],
                        zed_v.at[pl.ds(0, NLAST)])
        pltpu.sync_copy(zed_v.at[pl.ds(0, NLAST)],
                        den_out.at[pl.ds(nbase, NLAST)])


def _edge_b_body(hgr, pk_h, ex_h, den_h, out_h,
                 bufA, pk_v, ex_v, sidxA, sidxB, didxA, didxB,
                 sidx16, didx16, rowsA, rowsB, rows16, sem, acc_sh):
    c = lax.axis_index("c")
    s = lax.axis_index("s")
    ebase = pl.multiple_of(s * EPT, 8)
    nbase = pl.multiple_of(s * NSLICE, 8)

    pltpu.sync_copy(pk_h.at[pl.ds(ebase, EPT)], pk_v)
    pltpu.sync_copy(ex_h.at[pl.ds(ebase, EPT)], ex_v)
    pltpu.sync_copy(den_h, bufA)

    zero16 = jnp.zeros((16,), jnp.float32)

    def zr(r, _):
        for j in range(8):
            rowsA[r, pl.ds(j * 16, 16)] = zero16
        return 0
    lax.fori_loop(0, CHUNK, zr, 0)

    @pl.when(s < NTILE - 1)
    def _():
        for k in range(NSLICE // CHUNK):
            pltpu.sync_copy(rowsA,
                            acc_sh.at[pl.ds(nbase + k * CHUNK, CHUNK)])

    @pl.when(s == NTILE - 1)
    def _():
        for k in range(NLAST // CHUNK):
            pltpu.sync_copy(rowsA,
                            acc_sh.at[pl.ds(nbase + k * CHUNK, CHUNK)])
        pltpu.sync_copy(rowsA.at[pl.ds(0, NLAST % CHUNK)],
                        acc_sh.at[pl.ds(nbase + (NLAST // CHUNK) * CHUNK,
                                        NLAST % CHUNK)])

    plsc.subcore_barrier()

    cN = jnp.full((16,), c * N, jnp.int32)

    def build(g1, sidx_ref, didx_ref):
        def bj(j, _):
            off = g1 * CHUNK + j * 16
            pk = pk_v[pl.ds(off, 16)]
            didx = pk & 0xFFFF
            sidx_ref[pl.ds(j * 16, 16)] = (pk >> 16) + cN
            didx_ref[pl.ds(j * 16, 16)] = didx
            dn = plsc.load_gather(bufA, [didx])
            ex_v[pl.ds(off, 16)] = ex_v[pl.ds(off, 16)] / (dn + 1e-16)
            return 0
        lax.fori_loop(0, CHUNK // 16, bj, 0)

    def weight_scatter(g0, rows_ref, didx_ref, count):
        def wr(i, _):
            for u in range(2):
                r = i * 2 + u
                splat = plsc.load_gather(
                    ex_v, [jnp.full((16,), g0 * CHUNK + r, jnp.int32)])
                for j in range(8):
                    rows_ref[r, pl.ds(j * 16, 16)] = (
                        rows_ref[r, pl.ds(j * 16, 16)] * splat)
            return 0
        lax.fori_loop(0, count // 2, wr, 0)
        pltpu.sync_copy(rows_ref, acc_sh.at[didx_ref], add=True)

    build(0, sidxA, didxA)

    def pair(p, _):
        h = pltpu.async_copy(hgr.at[sidxA], rowsA, sem)

        @pl.when(p > 0)
        def _():
            weight_scatter(2 * p - 1, rowsB, didxB, CHUNK)
        build(2 * p + 1, sidxB, didxB)
        h.wait()

        h2 = pltpu.async_copy(hgr.at[sidxB], rowsB, sem)
        weight_scatter(2 * p, rowsA, didxA, CHUNK)

        @pl.when(p < NCHUNK // 2 - 1)
        def _():
            build(2 * p + 2, sidxA, didxA)
        h2.wait()
        return 0
    lax.fori_loop(0, NCHUNK // 2, pair, 0)
    weight_scatter(NCHUNK - 1, rowsB, didxB, CHUNK)

    toff = NCHUNK * CHUNK
    pk = pk_v[pl.ds(toff, 16)]
    didx = pk & 0xFFFF
    sidx16[...] = (pk >> 16) + cN
    didx16[...] = didx
    dn = plsc.load_gather(bufA, [didx])
    ex_v[pl.ds(toff, 16)] = ex_v[pl.ds(toff, 16)] / (dn + 1e-16)
    pltpu.async_copy(hgr.at[sidx16], rows16, sem).wait()

    def wrt(r, _):
        splat = plsc.load_gather(ex_v, [jnp.full((16,), toff + r, jnp.int32)])
        for j in range(8):
            rows16[r, pl.ds(j * 16, 16)] = rows16[r, pl.ds(j * 16, 16)] * splat
        return 0
    lax.fori_loop(0, TAIL, wrt, 0)
    pltpu.sync_copy(rows16, acc_sh.at[didx16], add=True)

    plsc.subcore_barrier()

    @pl.when(s < NTILE - 1)
    def _():
        pltpu.sync_copy(acc_sh.at[pl.ds(nbase, NSLICE)],
                        out_h.at[c, pl.ds(nbase, NSLICE)])

    @pl.when(s == NTILE - 1)
    def _():
        pltpu.sync_copy(acc_sh.at[pl.ds(nbase, NLAST)],
                        out_h.at[c, pl.ds(nbase, NLAST)])


def _edge_a(ss, sd, packed):
    return pl.kernel(
        _edge_a_body,
        out_type=[
            jax.ShapeDtypeStruct((E,), jnp.float32),
            jax.ShapeDtypeStruct((N,), jnp.float32),
        ],
        mesh=plsc.VectorSubcoreMesh(core_axis_name="c", subcore_axis_name="s"),
        compiler_params=pltpu.CompilerParams(needs_layout_passes=False),
        scratch_types=[
            pltpu.VMEM((N,), jnp.float32),
            pltpu.VMEM((N,), jnp.float32),
            pltpu.VMEM((EPT,), jnp.int32),
            pltpu.VMEM((EPT,), jnp.float32),
            pltpu.VMEM((128,), jnp.int32),
            pltpu.VMEM((16,), jnp.int32),
            pltpu.VMEM((NSLICE,), jnp.float32),
            pltpu.VMEM_SHARED((N,), jnp.float32),
        ],
    )(ss, sd, packed)


def _edge_b(hgr, packed, ex, den):
    return pl.kernel(
        _edge_b_body,
        out_type=jax.ShapeDtypeStruct((2, N, 128), jnp.float32),
        mesh=plsc.VectorSubcoreMesh(core_axis_name="c", subcore_axis_name="s"),
        compiler_params=pltpu.CompilerParams(needs_layout_passes=False),
        scratch_types=[
            pltpu.VMEM((N,), jnp.float32),
            pltpu.VMEM((EPT,), jnp.int32),
            pltpu.VMEM((EPT,), jnp.float32),
            pltpu.VMEM((CHUNK,), jnp.int32),
            pltpu.VMEM((CHUNK,), jnp.int32),
            pltpu.VMEM((CHUNK,), jnp.int32),
            pltpu.VMEM((CHUNK,), jnp.int32),
            pltpu.VMEM((16,), jnp.int32),
            pltpu.VMEM((16,), jnp.int32),
            pltpu.VMEM((CHUNK, 128), jnp.float32),
            pltpu.VMEM((CHUNK, 128), jnp.float32),
            pltpu.VMEM((16, 128), jnp.float32),
            pltpu.SemaphoreType.DMA,
            pltpu.VMEM_SHARED((N, 128), jnp.float32),
        ],
    )(hgr, packed, ex, den)


def _edge_phase(hgsplit, ss, sd, packed):
    ex, den = _edge_a(ss, sd, packed)
    return _edge_b(hgsplit.reshape(2 * N, 128), packed, ex, den)


def kernel(x, edges, W1, b1, g1, be1, W2, b2, g2, be2, W3, b3, g3, be3,
           Wg1, as1, ad1, bg1, Wg2, as2, ad2, bg2):
    packed = (edges[0] << 16) | edges[1]
    y1, st1 = _stem1(x, W1, b1)
    y2, st2 = _stem_mid(y1, st1, g1, be1, W2, b2)
    y3, st3 = _stem_mid(y2, st2, g2, be2, W3, b3)
    hg1, ss1, sd1 = _proj_bn(y3, st3, g3, be3, Wg1, as1, ad1)
    s1 = _edge_phase(hg1, ss1[:, 0], sd1[:, 0], packed)
    hg2, ss2, sd2 = _proj_bias(s1[0], s1[1], bg1, Wg2, as2, ad2)
    s2 = _edge_phase(hg2, ss2[:, 0], sd2[:, 0], packed)
    return _final(s2[0], s2[1], bg2)
